# baseline (speedup 1.0000x reference)
import functools

import jax
import jax.numpy as jnp
from jax import lax
from jax.experimental import pallas as pl
from jax.experimental.pallas import tpu as pltpu
from jax.experimental.pallas import tpu_sc as plsc

_DIM = 128
_NROWS = 11
_NC, _NS = 2, 16
_NW = _NC * _NS
_BLK = 400
_LANES = 16
_NV = _DIM // _LANES
_GRP = _BLK // _LANES


def _make_xe_kernel(n_edges):
    nblk = n_edges // _BLK
    nj = nblk // _NW

    mesh = plsc.VectorSubcoreMesh(
        core_axis_name="c", subcore_axis_name="s",
        num_cores=_NC, num_subcores=_NS,
    )

    @functools.partial(
        pl.kernel,
        out_type=jax.ShapeDtypeStruct((n_edges * _DIM,), jnp.float32),
        mesh=mesh,
        scratch_types=[
            pltpu.VMEM((_NROWS, _DIM), jnp.float32),
            pltpu.VMEM((_BLK,), jnp.int32),
            pltpu.VMEM((_BLK,), jnp.int32),
            pltpu.VMEM((_BLK * _DIM,), jnp.float32),
            pltpu.VMEM((_BLK * _DIM,), jnp.float32),
            pltpu.SemaphoreType.DMA,
            pltpu.SemaphoreType.DMA,
            pltpu.SemaphoreType.DMA,
            pltpu.SemaphoreType.DMA,
            pltpu.SemaphoreType.DMA,
            pltpu.SemaphoreType.DMA,
        ],
    )
    def xe_kernel(x_e, orders, btab, out, btab_v,
                  idxA, idxB, bufA, bufB, sxA, sxB, siA, siB, soA, soB):
        wid = lax.axis_index("s") * _NC + lax.axis_index("c")
        pltpu.sync_copy(btab, btab_v)

        def base_of(j):
            return (wid + _NW * j) * _BLK

        def in_x(j, buf, sem):
            return pltpu.make_async_copy(
                x_e.at[pl.ds(base_of(j) * _DIM, _BLK * _DIM)], buf, sem)

        def in_i(j, idx, sem):
            return pltpu.make_async_copy(
                orders.at[pl.ds(base_of(j), _BLK)], idx, sem)

        def out_c(j, buf, sem):
            return pltpu.make_async_copy(
                buf, out.at[pl.ds(base_of(j) * _DIM, _BLK * _DIM)], sem)

        def start_in(j, idx, buf, si, sx):
            in_i(j, idx, si).start()
            in_x(j, buf, sx).start()

        def wait_in(j, idx, buf, si, sx):
            in_i(j, idx, si).wait()
            in_x(j, buf, sx).wait()

        def compute(idx_v, buf):
            @plsc.parallel_loop(0, _GRP, unroll=2)
            def _(g):
                ovec = idx_v[pl.ds(g * _LANES, _LANES)]
                os_ = [ovec[r] for r in range(_LANES)]
                gbase = g * (_LANES * _DIM)

                def st(r, v, x):
                    plsc.addupdate(
                        buf.at[pl.ds(gbase + r * _DIM + v * _LANES,
                                     _LANES)], x)

                prev = [btab_v[os_[0], pl.ds(v * _LANES, _LANES)]
                        for v in range(_NV)]
                for r in range(1, _LANES):
                    o = os_[r]
                    cur = []
                    for v in range(_NV):
                        cur.append(btab_v[o, pl.ds(v * _LANES, _LANES)])
                        st(r - 1, v, prev[v])
                    prev = cur
                for v in range(_NV):
                    st(_LANES - 1, v, prev[v])

        start_in(0, idxA, bufA, siA, sxA)

        @pl.loop(0, nj - 1, step=2)
        def _(j):
            @pl.when(j > 0)
            def _():
                out_c(j - 1, bufB, soB).wait()

            start_in(j + 1, idxB, bufB, siB, sxB)
            wait_in(j, idxA, bufA, siA, sxA)
            compute(idxA, bufA)
            out_c(j, bufA, soA).start()
            wait_in(j + 1, idxB, bufB, siB, sxB)
            compute(idxB, bufB)
            out_c(j + 1, bufB, soB).start()
            out_c(j, bufA, soA).wait()

            @pl.when(j + 2 < nj)
            def _():
                start_in(j + 2, idxA, bufA, siA, sxA)

        jl = nj - 1
        out_c(jl - 1, bufB, soB).wait()
        wait_in(jl, idxA, bufA, siA, sxA)
        compute(idxA, bufA)
        out_c(jl, bufA, soA).start()
        out_c(jl, bufA, soA).wait()

    return xe_kernel


def _xv_body(xv_ref, b_ref, out_ref):
    out_ref[...] = xv_ref[...] + b_ref[1:2, :]


def _xv_add(x_v, b_table):
    n = x_v.shape[0]
    blk = 2000
    return pl.pallas_call(
        _xv_body,
        out_shape=jax.ShapeDtypeStruct((n, _DIM), jnp.float32),
        in_specs=[
            pl.BlockSpec((blk, _DIM), lambda i: (i, 0)),
            pl.BlockSpec((_NROWS, _DIM), lambda i: (0, 0)),
        ],
        out_specs=pl.BlockSpec((blk, _DIM), lambda i: (i, 0)),
        grid=(n // blk,),
    )(x_v, b_table)


def kernel(x_v, x_e, edge_orders, b_table):
    n_edges = x_e.shape[0]
    xe_flat = _make_xe_kernel(n_edges)(
        x_e.reshape(-1), edge_orders, b_table)
    xv_out = _xv_add(x_v, b_table)
    return (xv_out, xe_flat.reshape(n_edges, _DIM))


# speedup vs baseline: 1.0412x; 1.0412x over previous
import functools

import jax
import jax.numpy as jnp
from jax import lax
from jax.experimental import pallas as pl
from jax.experimental.pallas import tpu as pltpu
from jax.experimental.pallas import tpu_sc as plsc

_DIM = 128
_NROWS = 11
_NC, _NS = 2, 16
_NW = _NC * _NS
_BLK = 400
_LANES = 16
_NV = _DIM // _LANES
_GRP = _BLK // _LANES


def _make_xe_kernel(n_edges, n_nodes):
    nblk = n_edges // _BLK
    nj = nblk // _NW
    nvb = n_nodes // _BLK

    mesh = plsc.VectorSubcoreMesh(
        core_axis_name="c", subcore_axis_name="s",
        num_cores=_NC, num_subcores=_NS,
    )

    @functools.partial(
        pl.kernel,
        out_type=(
            jax.ShapeDtypeStruct((n_edges * _DIM,), jnp.float32),
            jax.ShapeDtypeStruct((n_nodes * _DIM,), jnp.float32),
        ),
        mesh=mesh,
        scratch_types=[
            pltpu.VMEM((_NROWS, _DIM), jnp.float32),
            pltpu.VMEM((_BLK,), jnp.int32),
            pltpu.VMEM((_BLK,), jnp.int32),
            pltpu.VMEM((_BLK * _DIM,), jnp.float32),
            pltpu.VMEM((_BLK * _DIM,), jnp.float32),
            pltpu.SemaphoreType.DMA,
            pltpu.SemaphoreType.DMA,
            pltpu.SemaphoreType.DMA,
            pltpu.SemaphoreType.DMA,
            pltpu.SemaphoreType.DMA,
            pltpu.SemaphoreType.DMA,
        ],
    )
    def xe_kernel(x_e, orders, btab, x_v, out, out_v, btab_v,
                  idxA, idxB, bufA, bufB, sxA, sxB, siA, siB, soA, soB):
        wid = lax.axis_index("s") * _NC + lax.axis_index("c")
        pltpu.sync_copy(btab, btab_v)

        def base_of(j):
            return (wid + _NW * j) * _BLK

        def in_x(j, buf, sem):
            return pltpu.make_async_copy(
                x_e.at[pl.ds(base_of(j) * _DIM, _BLK * _DIM)], buf, sem)

        def in_i(j, idx, sem):
            return pltpu.make_async_copy(
                orders.at[pl.ds(base_of(j), _BLK)], idx, sem)

        def out_c(j, buf, sem):
            return pltpu.make_async_copy(
                buf, out.at[pl.ds(base_of(j) * _DIM, _BLK * _DIM)], sem)

        def start_in(j, idx, buf, si, sx):
            in_i(j, idx, si).start()
            in_x(j, buf, sx).start()

        def wait_in(j, idx, buf, si, sx):
            in_i(j, idx, si).wait()
            in_x(j, buf, sx).wait()

        def compute(idx_v, buf):
            @plsc.parallel_loop(0, _GRP)
            def _(g):
                ovec = idx_v[pl.ds(g * _LANES, _LANES)]
                os_ = [ovec[r] for r in range(_LANES)]
                gbase = g * (_LANES * _DIM)

                def st(r, v, x):
                    plsc.addupdate(
                        buf.at[pl.ds(gbase + r * _DIM + v * _LANES,
                                     _LANES)], x)

                prev = [btab_v[os_[0], pl.ds(v * _LANES, _LANES)]
                        for v in range(_NV)]
                for r in range(1, _LANES):
                    o = os_[r]
                    cur = []
                    for v in range(_NV):
                        cur.append(btab_v[o, pl.ds(v * _LANES, _LANES)])
                        st(r - 1, v, prev[v])
                    prev = cur
                for v in range(_NV):
                    st(_LANES - 1, v, prev[v])

        start_in(0, idxA, bufA, siA, sxA)

        @pl.loop(0, nj - 1, step=2)
        def _(j):
            @pl.when(j > 0)
            def _():
                out_c(j - 1, bufB, soB).wait()

            start_in(j + 1, idxB, bufB, siB, sxB)
            wait_in(j, idxA, bufA, siA, sxA)
            compute(idxA, bufA)
            out_c(j, bufA, soA).start()
            wait_in(j + 1, idxB, bufB, siB, sxB)
            compute(idxB, bufB)
            out_c(j + 1, bufB, soB).start()
            out_c(j, bufA, soA).wait()

            @pl.when(j + 2 < nj)
            def _():
                start_in(j + 2, idxA, bufA, siA, sxA)

        jl = nj - 1
        out_c(jl - 1, bufB, soB).wait()
        wait_in(jl, idxA, bufA, siA, sxA)
        compute(idxA, bufA)
        out_c(jl, bufA, soA).start()
        out_c(jl, bufA, soA).wait()

        @pl.when(wid < nvb)
        def _():
            vbase = wid * _BLK * _DIM
            cp_v = pltpu.make_async_copy(
                x_v.at[pl.ds(vbase, _BLK * _DIM)], bufA, sxA)
            cp_v.start()
            b1 = [btab_v[1, pl.ds(v * _LANES, _LANES)] for v in range(_NV)]
            cp_v.wait()

            @plsc.parallel_loop(0, _BLK)
            def _(r):
                for v in range(_NV):
                    plsc.addupdate(
                        bufA.at[pl.ds(r * _DIM + v * _LANES, _LANES)], b1[v])

            cp_o = pltpu.make_async_copy(
                bufA, out_v.at[pl.ds(vbase, _BLK * _DIM)], soA)
            cp_o.start()
            cp_o.wait()

    return xe_kernel


def _xv_body(xv_ref, b_ref, out_ref):
    out_ref[...] = xv_ref[...] + b_ref[1:2, :]


def _xv_add(x_v, b_table):
    n = x_v.shape[0]
    blk = 2000
    return pl.pallas_call(
        _xv_body,
        out_shape=jax.ShapeDtypeStruct((n, _DIM), jnp.float32),
        in_specs=[
            pl.BlockSpec((blk, _DIM), lambda i: (i, 0)),
            pl.BlockSpec((_NROWS, _DIM), lambda i: (0, 0)),
        ],
        out_specs=pl.BlockSpec((blk, _DIM), lambda i: (i, 0)),
        grid=(n // blk,),
    )(x_v, b_table)


def kernel(x_v, x_e, edge_orders, b_table):
    n_edges = x_e.shape[0]
    n_nodes = x_v.shape[0]
    xe_flat, xv_flat = _make_xe_kernel(n_edges, n_nodes)(
        x_e.reshape(-1), edge_orders, b_table, x_v.reshape(-1))
    return (xv_flat.reshape(n_nodes, _DIM), xe_flat.reshape(n_edges, _DIM))


# speedup vs baseline: 1.0763x; 1.0338x over previous
import functools

import jax
import jax.numpy as jnp
from jax import lax
from jax.experimental import pallas as pl
from jax.experimental.pallas import tpu as pltpu
from jax.experimental.pallas import tpu_sc as plsc

_DIM = 128
_NROWS = 11
_NC, _NS = 2, 16
_NW = _NC * _NS
_BLK = 400
_LANES = 16
_NV = _DIM // _LANES
_GRP = _BLK // _LANES


def _make_xe_kernel(n_edges):
    nblk = n_edges // _BLK
    nj = nblk // _NW

    mesh = plsc.VectorSubcoreMesh(
        core_axis_name="c", subcore_axis_name="s",
        num_cores=_NC, num_subcores=_NS,
    )

    @functools.partial(
        pl.kernel,
        out_type=jax.ShapeDtypeStruct((n_edges * _DIM,), jnp.float32),
        mesh=mesh,
        scratch_types=[
            pltpu.VMEM((_NROWS, _DIM), jnp.float32),
            pltpu.VMEM((_BLK,), jnp.int32),
            pltpu.VMEM((_BLK,), jnp.int32),
            pltpu.VMEM((_BLK * _DIM,), jnp.float32),
            pltpu.VMEM((_BLK * _DIM,), jnp.float32),
            pltpu.SemaphoreType.DMA,
            pltpu.SemaphoreType.DMA,
            pltpu.SemaphoreType.DMA,
            pltpu.SemaphoreType.DMA,
            pltpu.SemaphoreType.DMA,
            pltpu.SemaphoreType.DMA,
        ],
    )
    def xe_kernel(x_e, orders, btab, out, btab_v,
                  idxA, idxB, bufA, bufB, sxA, sxB, siA, siB, soA, soB):
        wid = lax.axis_index("s") * _NC + lax.axis_index("c")
        pltpu.sync_copy(btab, btab_v)

        def base_of(j):
            return (wid + _NW * j) * _BLK

        def in_x(j, buf, sem):
            return pltpu.make_async_copy(
                x_e.at[pl.ds(base_of(j) * _DIM, _BLK * _DIM)], buf, sem)

        def in_i(j, idx, sem):
            return pltpu.make_async_copy(
                orders.at[pl.ds(base_of(j), _BLK)], idx, sem)

        def out_c(j, buf, sem):
            return pltpu.make_async_copy(
                buf, out.at[pl.ds(base_of(j) * _DIM, _BLK * _DIM)], sem)

        def start_in(j, idx, buf, si, sx):
            in_i(j, idx, si).start()
            in_x(j, buf, sx).start()

        def wait_in(j, idx, buf, si, sx):
            in_i(j, idx, si).wait()
            in_x(j, buf, sx).wait()

        def compute(idx_v, buf):
            @plsc.parallel_loop(0, _GRP)
            def _(g):
                ovec = idx_v[pl.ds(g * _LANES, _LANES)]
                os_ = [ovec[r] for r in range(_LANES)]
                gbase = g * (_LANES * _DIM)

                def st(r, v, x):
                    plsc.addupdate(
                        buf.at[pl.ds(gbase + r * _DIM + v * _LANES,
                                     _LANES)], x)

                prev = [btab_v[os_[0], pl.ds(v * _LANES, _LANES)]
                        for v in range(_NV)]
                for r in range(1, _LANES):
                    o = os_[r]
                    cur = []
                    for v in range(_NV):
                        cur.append(btab_v[o, pl.ds(v * _LANES, _LANES)])
                        st(r - 1, v, prev[v])
                    prev = cur
                for v in range(_NV):
                    st(_LANES - 1, v, prev[v])

        start_in(0, idxA, bufA, siA, sxA)

        @pl.loop(0, nj - 1, step=2)
        def _(j):
            @pl.when(j > 0)
            def _():
                out_c(j - 1, bufB, soB).wait()

            start_in(j + 1, idxB, bufB, siB, sxB)
            wait_in(j, idxA, bufA, siA, sxA)
            compute(idxA, bufA)
            out_c(j, bufA, soA).start()
            wait_in(j + 1, idxB, bufB, siB, sxB)
            compute(idxB, bufB)
            out_c(j + 1, bufB, soB).start()
            out_c(j, bufA, soA).wait()

            @pl.when(j + 2 < nj)
            def _():
                start_in(j + 2, idxA, bufA, siA, sxA)

        jl = nj - 1
        out_c(jl - 1, bufB, soB).wait()
        wait_in(jl, idxA, bufA, siA, sxA)
        compute(idxA, bufA)
        out_c(jl, bufA, soA).start()
        out_c(jl, bufA, soA).wait()

    return xe_kernel


def _xv_body(xv_ref, b_ref, out_ref):
    out_ref[...] = xv_ref[...] + b_ref[1:2, :]


def _xv_add(x_v, b_table):
    n = x_v.shape[0]
    blk = 2000
    return pl.pallas_call(
        _xv_body,
        out_shape=jax.ShapeDtypeStruct((n, _DIM), jnp.float32),
        in_specs=[
            pl.BlockSpec((blk, _DIM), lambda i: (i, 0)),
            pl.BlockSpec((_NROWS, _DIM), lambda i: (0, 0)),
        ],
        out_specs=pl.BlockSpec((blk, _DIM), lambda i: (i, 0)),
        grid=(n // blk,),
    )(x_v, b_table)


def kernel(x_v, x_e, edge_orders, b_table):
    n_edges = x_e.shape[0]
    xe_flat = _make_xe_kernel(n_edges)(
        x_e.reshape(-1), edge_orders, b_table)
    xv_out = _xv_add(x_v, b_table)
    return (xv_out, xe_flat.reshape(n_edges, _DIM))
